# baseline (speedup 1.0000x reference)
import functools

import jax
import jax.numpy as jnp
from jax import lax
from jax.experimental import pallas as pl
from jax.experimental.pallas import tpu as pltpu
from jax.experimental.pallas import tpu_sc as plsc

N_NODES = 10000
N_EDGES = 320000
D = 128

NC = 2
NS = 16
NW = NC * NS

CH = 128
EPT = N_EDGES // NW
CHUNKS = 79
EPT_PAD = CHUNKS * CH
NP = 10112
ROWS_PT = NP // NS

_mesh = plsc.VectorSubcoreMesh(
    core_axis_name="c", subcore_axis_name="s", num_cores=NC, num_subcores=NS)


DEG_W = D


@functools.partial(
    pl.kernel,
    out_type=jax.ShapeDtypeStruct((NC, NP, DEG_W), jnp.float32),
    mesh=_mesh,
    scratch_types=[
        pltpu.VMEM((CHUNKS, CH), jnp.int32),
        pltpu.VMEM((CH, DEG_W), jnp.float32),
        pltpu.VMEM_SHARED((NP, DEG_W), jnp.float32),
    ],
)
def _deg_kernel(dst_hbm, ones_hbm, zeros_hbm, out_hbm, dst_v, ones_v, hist_sh):
    c = lax.axis_index("c")
    s = lax.axis_index("s")
    wid = c * NS + s
    pltpu.sync_copy(zeros_hbm.at[pl.ds(s * ROWS_PT, ROWS_PT)],
                    hist_sh.at[pl.ds(s * ROWS_PT, ROWS_PT)])
    pltpu.sync_copy(ones_hbm, ones_v)
    pltpu.sync_copy(dst_hbm.at[wid], dst_v)
    plsc.subcore_barrier()

    def body(j, carry):
        pltpu.sync_copy(ones_v, hist_sh.at[dst_v.at[j]], add=True)
        return carry

    lax.fori_loop(0, CHUNKS, body, 0)
    plsc.subcore_barrier()
    pltpu.sync_copy(hist_sh.at[pl.ds(s * ROWS_PT, ROWS_PT)],
                    out_hbm.at[c, pl.ds(s * ROWS_PT, ROWS_PT)])


@functools.partial(
    pl.kernel,
    out_type=jax.ShapeDtypeStruct((NC, NP, D), jnp.float32),
    mesh=_mesh,
    scratch_types=[
        pltpu.VMEM((CHUNKS, CH), jnp.int32),
        pltpu.VMEM((CHUNKS, CH), jnp.int32),
        pltpu.VMEM((CH, D), jnp.float32),
        pltpu.VMEM_SHARED((NP, D), jnp.float32),
        pltpu.SemaphoreType.DMA,
    ],
)
def _agg_kernel(y_hbm, src_hbm, dst_hbm, zeros_hbm, out_hbm,
                src_v, dst_v, rows_v, z_sh, sem):
    c = lax.axis_index("c")
    s = lax.axis_index("s")
    wid = c * NS + s
    pltpu.sync_copy(zeros_hbm.at[pl.ds(s * ROWS_PT, ROWS_PT)],
                    z_sh.at[pl.ds(s * ROWS_PT, ROWS_PT)])
    pltpu.sync_copy(src_hbm.at[wid], src_v)
    pltpu.sync_copy(dst_hbm.at[wid], dst_v)
    plsc.subcore_barrier()

    def body(j, carry):
        pltpu.async_copy(y_hbm.at[src_v.at[j]], rows_v, sem).wait()
        pltpu.sync_copy(rows_v, z_sh.at[dst_v.at[j]], add=True)
        return carry

    lax.fori_loop(0, CHUNKS, body, 0)
    plsc.subcore_barrier()
    pltpu.sync_copy(z_sh.at[pl.ds(s * ROWS_PT, ROWS_PT)],
                    out_hbm.at[c, pl.ds(s * ROWS_PT, ROWS_PT)])


def _tc1_body(x_ref, w_ref, hist_ref, y_ref, dinv_ref):
    deg = 1.0 + hist_ref[0, :, 0:1] + hist_ref[1, :, 0:1]
    dinv = lax.rsqrt(deg)
    y = jnp.dot(x_ref[...], w_ref[...], preferred_element_type=jnp.float32)
    y_ref[...] = y * dinv
    dinv_ref[...] = dinv


def _tc1(x_pad, w1, hist):
    return pl.pallas_call(
        _tc1_body,
        out_shape=(jax.ShapeDtypeStruct((NP, D), jnp.float32),
                   jax.ShapeDtypeStruct((NP, 1), jnp.float32)),
    )(x_pad, w1, hist)


def _tc2_body(zp_ref, y_ref, dinv_ref, w_ref, b_ref, y2_ref):
    dinv = dinv_ref[...]
    h = dinv * (zp_ref[0] + zp_ref[1] + y_ref[...]) + b_ref[...]
    h = jnp.maximum(h, 0.0)
    rows = lax.broadcasted_iota(jnp.int32, (NP, D), 0)
    h = jnp.where(rows < N_NODES, h, 0.0)
    y2 = jnp.dot(h, w_ref[...], preferred_element_type=jnp.float32)
    y2_ref[...] = y2 * dinv


def _tc2(zp, y1, dinv, w2, b1):
    return pl.pallas_call(
        _tc2_body,
        out_shape=jax.ShapeDtypeStruct((NP, D), jnp.float32),
    )(zp, y1, dinv, w2, b1)


def _tc3_body(zp_ref, y_ref, dinv_ref, b_ref, fw1_ref, fb1_ref,
              fw2_ref, fb2_ref, p1_ref, p2_ref):
    h = dinv_ref[...] * (zp_ref[0] + zp_ref[1] + y_ref[...]) + b_ref[...]
    h = jnp.maximum(h, 0.0)
    rows = lax.broadcasted_iota(jnp.int32, (NP, D), 0)
    h = jnp.where(rows < N_NODES, h, 0.0)
    hbar = jnp.sum(h, axis=0, keepdims=True) * (1.0 / N_NODES)
    l1 = jnp.dot(hbar, fw1_ref[...], preferred_element_type=jnp.float32) + fb1_ref[...]
    l2 = jnp.dot(hbar, fw2_ref[...], preferred_element_type=jnp.float32) + fb2_ref[...]
    e1 = jnp.exp(l1 - jnp.max(l1, axis=-1, keepdims=True))
    e2 = jnp.exp(l2 - jnp.max(l2, axis=-1, keepdims=True))
    p1_ref[...] = e1 / jnp.sum(e1, axis=-1, keepdims=True)
    p2_ref[...] = e2 / jnp.sum(e2, axis=-1, keepdims=True)


def _tc3(zp, y2, dinv, b2, fw1, fb1, fw2, fb2):
    return pl.pallas_call(
        _tc3_body,
        out_shape=(jax.ShapeDtypeStruct((1, 64), jnp.float32),
                   jax.ShapeDtypeStruct((1, 32), jnp.float32)),
    )(zp, y2, dinv, b2, fw1, fb1, fw2, fb2)


def kernel(x, edge_index, W1, b1, W2, b2, fcW1, fcb1, fcW2, fcb2):
    src = edge_index[0].astype(jnp.int32)
    dst = edge_index[1].astype(jnp.int32)
    pad = jnp.full((NW, EPT_PAD - EPT), N_NODES, jnp.int32)
    src3 = jnp.concatenate([src.reshape(NW, EPT), pad], axis=1).reshape(NW, CHUNKS, CH)
    dst3 = jnp.concatenate([dst.reshape(NW, EPT), pad], axis=1).reshape(NW, CHUNKS, CH)

    x_pad = jnp.zeros((NP, D), jnp.float32).at[:N_NODES].set(x)
    onesW = jnp.ones((CH, DEG_W), jnp.float32)
    zerosW = jnp.zeros((NP, DEG_W), jnp.float32)
    zerosD = jnp.zeros((NP, D), jnp.float32)

    hist = _deg_kernel(dst3, onesW, zerosW)
    y1, dinv = _tc1(x_pad, W1, hist)
    zp1 = _agg_kernel(y1, src3, dst3, zerosD)
    y2 = _tc2(zp1, y1, dinv, W2, b1.reshape(1, D))
    zp2 = _agg_kernel(y2, src3, dst3, zerosD)
    p1, p2 = _tc3(zp2, y2, dinv, b2.reshape(1, D),
                  fcW1, fcb1.reshape(1, 64), fcW2, fcb2.reshape(1, 32))
    return (p1.reshape(64), p2.reshape(32))


# speedup vs baseline: 1.0011x; 1.0011x over previous
import functools

import jax
import jax.numpy as jnp
from jax import lax
from jax.experimental import pallas as pl
from jax.experimental.pallas import tpu as pltpu
from jax.experimental.pallas import tpu_sc as plsc

N_NODES = 10000
N_EDGES = 320000
D = 128

NC = 2
NS = 16
NW = NC * NS

CH = 128
EPT = N_EDGES // NW
CHUNKS = 79
EPT_PAD = CHUNKS * CH
NP = 10112
ROWS_PT = NP // NS

DEG_W = D


@functools.cache
def _sc_kernels():
    mesh = plsc.VectorSubcoreMesh(
        core_axis_name="c", subcore_axis_name="s",
        num_cores=NC, num_subcores=NS)

    @functools.partial(
        pl.kernel,
        out_type=jax.ShapeDtypeStruct((NC, NP, DEG_W), jnp.float32),
        mesh=mesh,
        scratch_types=[
            pltpu.VMEM((CHUNKS, CH), jnp.int32),
            pltpu.VMEM((CH, DEG_W), jnp.float32),
            pltpu.VMEM_SHARED((NP, DEG_W), jnp.float32),
        ],
    )
    def deg_kernel(dst_hbm, ones_hbm, zeros_hbm, out_hbm,
                   dst_v, ones_v, hist_sh):
        c = lax.axis_index("c")
        s = lax.axis_index("s")
        wid = c * NS + s
        pltpu.sync_copy(zeros_hbm.at[pl.ds(s * ROWS_PT, ROWS_PT)],
                        hist_sh.at[pl.ds(s * ROWS_PT, ROWS_PT)])
        pltpu.sync_copy(ones_hbm, ones_v)
        pltpu.sync_copy(dst_hbm.at[wid], dst_v)
        plsc.subcore_barrier()

        def body(j, carry):
            pltpu.sync_copy(ones_v, hist_sh.at[dst_v.at[j]], add=True)
            return carry

        lax.fori_loop(0, CHUNKS, body, 0)
        plsc.subcore_barrier()
        pltpu.sync_copy(hist_sh.at[pl.ds(s * ROWS_PT, ROWS_PT)],
                        out_hbm.at[c, pl.ds(s * ROWS_PT, ROWS_PT)])

    @functools.partial(
        pl.kernel,
        out_type=jax.ShapeDtypeStruct((NC, NP, D), jnp.float32),
        mesh=mesh,
        scratch_types=[
            pltpu.VMEM((CHUNKS, CH), jnp.int32),
            pltpu.VMEM((CHUNKS, CH), jnp.int32),
            pltpu.VMEM((CH, D), jnp.float32),
            pltpu.VMEM_SHARED((NP, D), jnp.float32),
            pltpu.SemaphoreType.DMA,
        ],
    )
    def agg_kernel(y_hbm, src_hbm, dst_hbm, zeros_hbm, out_hbm,
                   src_v, dst_v, rows_v, z_sh, sem):
        c = lax.axis_index("c")
        s = lax.axis_index("s")
        wid = c * NS + s
        pltpu.sync_copy(zeros_hbm.at[pl.ds(s * ROWS_PT, ROWS_PT)],
                        z_sh.at[pl.ds(s * ROWS_PT, ROWS_PT)])
        pltpu.sync_copy(src_hbm.at[wid], src_v)
        pltpu.sync_copy(dst_hbm.at[wid], dst_v)
        plsc.subcore_barrier()

        def body(j, carry):
            pltpu.async_copy(y_hbm.at[src_v.at[j]], rows_v, sem).wait()
            pltpu.sync_copy(rows_v, z_sh.at[dst_v.at[j]], add=True)
            return carry

        lax.fori_loop(0, CHUNKS, body, 0)
        plsc.subcore_barrier()
        pltpu.sync_copy(z_sh.at[pl.ds(s * ROWS_PT, ROWS_PT)],
                        out_hbm.at[c, pl.ds(s * ROWS_PT, ROWS_PT)])

    return deg_kernel, agg_kernel


def _tc1_body(x_ref, w_ref, hist_ref, y_ref, dinv_ref):
    deg = 1.0 + hist_ref[0, :, 0:1] + hist_ref[1, :, 0:1]
    dinv = lax.rsqrt(deg)
    y = jnp.dot(x_ref[...], w_ref[...], preferred_element_type=jnp.float32)
    y_ref[...] = y * dinv
    dinv_ref[...] = dinv


def _tc1(x_pad, w1, hist):
    return pl.pallas_call(
        _tc1_body,
        out_shape=(jax.ShapeDtypeStruct((NP, D), jnp.float32),
                   jax.ShapeDtypeStruct((NP, 1), jnp.float32)),
    )(x_pad, w1, hist)


def _tc2_body(zp_ref, y_ref, dinv_ref, w_ref, b_ref, y2_ref):
    dinv = dinv_ref[...]
    h = dinv * (zp_ref[0] + zp_ref[1] + y_ref[...]) + b_ref[...]
    h = jnp.maximum(h, 0.0)
    rows = lax.broadcasted_iota(jnp.int32, (NP, D), 0)
    h = jnp.where(rows < N_NODES, h, 0.0)
    y2 = jnp.dot(h, w_ref[...], preferred_element_type=jnp.float32)
    y2_ref[...] = y2 * dinv


def _tc2(zp, y1, dinv, w2, b1):
    return pl.pallas_call(
        _tc2_body,
        out_shape=jax.ShapeDtypeStruct((NP, D), jnp.float32),
    )(zp, y1, dinv, w2, b1)


def _tc3_body(zp_ref, y_ref, dinv_ref, b_ref, fw1_ref, fb1_ref,
              fw2_ref, fb2_ref, p1_ref, p2_ref):
    h = dinv_ref[...] * (zp_ref[0] + zp_ref[1] + y_ref[...]) + b_ref[...]
    h = jnp.maximum(h, 0.0)
    rows = lax.broadcasted_iota(jnp.int32, (NP, D), 0)
    h = jnp.where(rows < N_NODES, h, 0.0)
    hbar = jnp.sum(h, axis=0, keepdims=True) * (1.0 / N_NODES)
    l1 = jnp.dot(hbar, fw1_ref[...], preferred_element_type=jnp.float32) + fb1_ref[...]
    l2 = jnp.dot(hbar, fw2_ref[...], preferred_element_type=jnp.float32) + fb2_ref[...]
    e1 = jnp.exp(l1 - jnp.max(l1, axis=-1, keepdims=True))
    e2 = jnp.exp(l2 - jnp.max(l2, axis=-1, keepdims=True))
    p1_ref[...] = e1 / jnp.sum(e1, axis=-1, keepdims=True)
    p2_ref[...] = e2 / jnp.sum(e2, axis=-1, keepdims=True)


def _tc3(zp, y2, dinv, b2, fw1, fb1, fw2, fb2):
    return pl.pallas_call(
        _tc3_body,
        out_shape=(jax.ShapeDtypeStruct((1, 64), jnp.float32),
                   jax.ShapeDtypeStruct((1, 32), jnp.float32)),
    )(zp, y2, dinv, b2, fw1, fb1, fw2, fb2)


def kernel(x, edge_index, W1, b1, W2, b2, fcW1, fcb1, fcW2, fcb2):
    src = edge_index[0].astype(jnp.int32)
    dst = edge_index[1].astype(jnp.int32)
    pad = jnp.full((NW, EPT_PAD - EPT), N_NODES, jnp.int32)
    src3 = jnp.concatenate([src.reshape(NW, EPT), pad], axis=1).reshape(NW, CHUNKS, CH)
    dst3 = jnp.concatenate([dst.reshape(NW, EPT), pad], axis=1).reshape(NW, CHUNKS, CH)

    x_pad = jnp.zeros((NP, D), jnp.float32).at[:N_NODES].set(x)
    onesW = jnp.ones((CH, DEG_W), jnp.float32)
    zerosD = jnp.zeros((NP, D), jnp.float32)

    deg_kernel, agg_kernel = _sc_kernels()
    hist = deg_kernel(dst3, onesW, zerosD)
    y1, dinv = _tc1(x_pad, W1, hist)
    zp1 = agg_kernel(y1, src3, dst3, zerosD)
    y2 = _tc2(zp1, y1, dinv, W2, b1.reshape(1, D))
    zp2 = agg_kernel(y2, src3, dst3, zerosD)
    p1, p2 = _tc3(zp2, y2, dinv, b2.reshape(1, D),
                  fcW1, fcb1.reshape(1, 64), fcW2, fcb2.reshape(1, 32))
    return (p1.reshape(64), p2.reshape(32))
